# baseline (speedup 1.0000x reference)
import functools

import jax
import jax.numpy as jnp
from jax import lax
from jax.experimental import pallas as pl
from jax.experimental.pallas import tpu as pltpu
from jax.experimental.pallas import tpu_sc as plsc

N = 100000
E = 1600000
H = 16

NC = 2
NS = 16
NW = NC * NS

SUB = 128
GRP = 4
CHUNK = SUB * GRP
NCH = E // CHUNK
CH_HI = 98
CH_LO = 97
W_HI = NCH - NW * CH_LO
IDX_ROWS = E // SUB

RPT = N // NS
ZROWS = 125
ZCOPIES = RPT // ZROWS

_mesh = plsc.VectorSubcoreMesh(core_axis_name="c", subcore_axis_name="s",
                               num_cores=NC, num_subcores=NS)
_sc_params = pltpu.CompilerParams(use_tc_tiling_on_sc=False,
                                  needs_layout_passes=False)


def _sds(shape):
    return jax.ShapeDtypeStruct(shape, jnp.float32)


def _worker_chunks(wid):
    nb = jnp.where(wid < W_HI, CH_HI, CH_LO)
    cbase = jnp.where(wid < W_HI, wid * CH_HI,
                      W_HI * CH_HI + (wid - W_HI) * CH_LO)
    return nb, cbase


def _zero_acc(acc, zb, sem, my_rows):
    ds = [pltpu.async_copy(zb, acc.at[pl.ds(my_rows + j * ZROWS, ZROWS)], sem)
          for j in range(ZCOPIES)]
    for d in ds:
        d.wait()


@functools.partial(
    pl.kernel,
    out_type=(_sds((NC, N, H)), _sds((NC, N, H))),
    mesh=_mesh,
    scratch_types=[
        pltpu.VMEM((GRP, SUB), jnp.int32),
        pltpu.VMEM((H, CHUNK), jnp.float32),
        pltpu.VMEM((CHUNK, H), jnp.float32),
        pltpu.VMEM((ZROWS, H), jnp.float32),
        pltpu.VMEM_SHARED((N, H), jnp.float32),
        pltpu.SemaphoreType.DMA,
    ],
    compiler_params=_sc_params,
)
def _sc_scatter0(ei_hbm, a0t_hbm, s0_out, cnt_out, idxd, a0t, rows, zb, acc,
                 sem):
    c = lax.axis_index("c")
    s = lax.axis_index("s")
    wid = s * NC + c
    nb, cbase = _worker_chunks(wid)
    my_rows = s * RPT
    my_slice = pl.ds(my_rows, RPT)
    lanes = lax.iota(jnp.int32, H)

    @plsc.parallel_loop(0, ZROWS, unroll=8)
    def _(i):
        zb[i, :] = jnp.zeros((H,), jnp.float32)

    _zero_acc(acc, zb, sem, my_rows)
    plsc.subcore_barrier()

    def chunk_a(i, _):
        g = cbase + i
        pltpu.sync_copy(ei_hbm.at[1, pl.ds(g * GRP, GRP)], idxd)
        pltpu.async_copy(a0t_hbm.at[:, pl.ds(g * CHUNK, CHUNK)], a0t,
                         sem).wait()

        @plsc.parallel_loop(0, CHUNK, unroll=8)
        def _(k):
            rows[k, :] = plsc.load_gather(a0t, [lanes, jnp.full((H,), k,
                                                                jnp.int32)])

        for j in range(GRP):
            pltpu.sync_copy(rows.at[pl.ds(j * SUB, SUB)],
                            acc.at[idxd.at[j]], add=True)
        return 0

    lax.fori_loop(0, nb, chunk_a, 0)
    plsc.subcore_barrier()
    pltpu.sync_copy(acc.at[my_slice], s0_out.at[c, my_slice])
    plsc.subcore_barrier()

    _zero_acc(acc, zb, sem, my_rows)

    @plsc.parallel_loop(0, CHUNK, unroll=8)
    def _(i):
        rows[i, :] = jnp.ones((H,), jnp.float32)

    plsc.subcore_barrier()

    def chunk_b(i, _):
        g = cbase + i
        pltpu.sync_copy(ei_hbm.at[1, pl.ds(g * GRP, GRP)], idxd)
        for j in range(GRP):
            pltpu.sync_copy(rows.at[pl.ds(j * SUB, SUB)],
                            acc.at[idxd.at[j]], add=True)
        return 0

    lax.fori_loop(0, nb, chunk_b, 0)
    plsc.subcore_barrier()
    pltpu.sync_copy(acc.at[my_slice], cnt_out.at[c, my_slice])


@functools.partial(
    pl.kernel,
    out_type=_sds((NC, N, H)),
    mesh=_mesh,
    scratch_types=[
        pltpu.VMEM((GRP, SUB), jnp.int32),
        pltpu.VMEM((GRP, SUB), jnp.int32),
        pltpu.VMEM((CHUNK, H), jnp.float32),
        pltpu.VMEM((CHUNK, H), jnp.float32),
        pltpu.VMEM((H, CHUNK), jnp.float32),
        pltpu.VMEM((ZROWS, H), jnp.float32),
        pltpu.VMEM_SHARED((N, H), jnp.float32),
        pltpu.SemaphoreType.DMA,
    ],
    compiler_params=_sc_params,
)
def _sc_layer(ei_hbm, p_hbm, q_hbm, rt_hbm, s_out,
              idxd, idxs, pb, qb, rt, zb, acc, sem):
    c = lax.axis_index("c")
    s = lax.axis_index("s")
    wid = s * NC + c
    nb, cbase = _worker_chunks(wid)
    my_rows = s * RPT
    my_slice = pl.ds(my_rows, RPT)
    lanes = lax.iota(jnp.int32, H)

    @plsc.parallel_loop(0, ZROWS, unroll=8)
    def _(i):
        zb[i, :] = jnp.zeros((H,), jnp.float32)

    _zero_acc(acc, zb, sem, my_rows)
    plsc.subcore_barrier()

    def chunk(i, _):
        g = cbase + i
        pltpu.sync_copy(ei_hbm.at[1, pl.ds(g * GRP, GRP)], idxd)
        pltpu.sync_copy(ei_hbm.at[0, pl.ds(g * GRP, GRP)], idxs)
        ds = [pltpu.async_copy(rt_hbm.at[:, pl.ds(g * CHUNK, CHUNK)], rt,
                               sem)]
        for j in range(GRP):
            sl = pl.ds(j * SUB, SUB)
            ds.append(pltpu.async_copy(p_hbm.at[idxd.at[j]], pb.at[sl], sem))
            ds.append(pltpu.async_copy(q_hbm.at[idxs.at[j]], qb.at[sl], sem))
        for d in ds:
            d.wait()

        @plsc.parallel_loop(0, CHUNK, unroll=8)
        def _(k):
            r_col = plsc.load_gather(rt, [lanes, jnp.full((H,), k, jnp.int32)])
            pb[k, :] = jnp.maximum(pb[k, :] + qb[k, :] + r_col, 0.0)

        for j in range(GRP):
            pltpu.sync_copy(pb.at[pl.ds(j * SUB, SUB)],
                            acc.at[idxd.at[j]], add=True)
        return 0

    lax.fori_loop(0, nb, chunk, 0)
    plsc.subcore_barrier()
    pltpu.sync_copy(acc.at[my_slice], s_out.at[c, my_slice])


EB = 12800
EBLOCKS = E // EB
NR = N // 8
NB = 512
NBLOCKS = -(-NR // NB)


def _t0_body(ea_ref, w_ref, b_ref, a0_ref, r0_ref, r1_ref):
    y = jnp.dot(w_ref[...], ea_ref[...], preferred_element_type=jnp.float32)
    y = y + b_ref[...]
    a0_ref[...] = jnp.maximum(y[:H], 0.0)
    r0_ref[...] = y[H:2 * H]
    r1_ref[...] = y[2 * H:]


def _edge_precompute(ea_t, wcat_t, bcol):
    return pl.pallas_call(
        _t0_body,
        grid=(EBLOCKS,),
        in_specs=[
            pl.BlockSpec((H, EB), lambda i: (0, i)),
            pl.BlockSpec((3 * H, H), lambda i: (0, 0)),
            pl.BlockSpec((3 * H, 1), lambda i: (0, 0)),
        ],
        out_specs=[pl.BlockSpec((H, EB), lambda i: (0, i))] * 3,
        out_shape=[_sds((H, E))] * 3,
    )(ea_t, wcat_t, bcol)


def _t1_body(s_ref, c_ref, w2_ref, b2_ref, wab_ref, p_ref, q_ref,
             cnt_ref, invd_ref):
    ssum = s_ref[0] + s_ref[1]
    cnt = c_ref[0] + c_ref[1]
    nf = jnp.dot(ssum, w2_ref[...], preferred_element_type=jnp.float32)
    nf = nf + cnt * b2_ref[...]
    pq = jnp.dot(nf, wab_ref[...], preferred_element_type=jnp.float32)
    p_ref[...] = pq[:, :128]
    q_ref[...] = pq[:, 128:]
    cnt_ref[...] = cnt
    invd_ref[...] = 1.0 / jnp.maximum(cnt, 1.0)


def _node_init(s0_parts, cnt_parts, w2bd, b2tile, wabbd):
    return pl.pallas_call(
        _t1_body,
        grid=(NBLOCKS,),
        in_specs=[
            pl.BlockSpec((NC, NB, 128), lambda i: (0, i, 0)),
            pl.BlockSpec((NC, NB, 128), lambda i: (0, i, 0)),
            pl.BlockSpec((128, 128), lambda i: (0, 0)),
            pl.BlockSpec((1, 128), lambda i: (0, 0)),
            pl.BlockSpec((128, 256), lambda i: (0, 0)),
        ],
        out_specs=[pl.BlockSpec((NB, 128), lambda i: (i, 0))] * 4,
        out_shape=[_sds((NR, 128))] * 4,
    )(s0_parts, cnt_parts, w2bd, b2tile, wabbd)


def _t2_body(s_ref, cnt_ref, invd_ref, ew2_ref, eb2_ref, nw_ref, nb_ref,
             wab_ref, p_ref, q_ref):
    ssum = s_ref[0] + s_ref[1]
    agg = jnp.dot(ssum, ew2_ref[...], preferred_element_type=jnp.float32)
    agg = (agg + cnt_ref[...] * eb2_ref[...]) * invd_ref[...]
    nf = jnp.dot(agg, nw_ref[...], preferred_element_type=jnp.float32)
    nf = jnp.maximum(nf + nb_ref[...], 0.0)
    pq = jnp.dot(nf, wab_ref[...], preferred_element_type=jnp.float32)
    p_ref[...] = pq[:, :128]
    q_ref[...] = pq[:, 128:]


def _node_update(s_parts, cnt, invd, ew2bd, eb2t, nwbd, nbt, wabbd):
    return pl.pallas_call(
        _t2_body,
        grid=(NBLOCKS,),
        in_specs=[
            pl.BlockSpec((NC, NB, 128), lambda i: (0, i, 0)),
            pl.BlockSpec((NB, 128), lambda i: (i, 0)),
            pl.BlockSpec((NB, 128), lambda i: (i, 0)),
            pl.BlockSpec((128, 128), lambda i: (0, 0)),
            pl.BlockSpec((1, 128), lambda i: (0, 0)),
            pl.BlockSpec((128, 128), lambda i: (0, 0)),
            pl.BlockSpec((1, 128), lambda i: (0, 0)),
            pl.BlockSpec((128, 256), lambda i: (0, 0)),
        ],
        out_specs=[pl.BlockSpec((NB, 128), lambda i: (i, 0))] * 2,
        out_shape=[_sds((NR, 128))] * 2,
    )(s_parts, cnt, invd, ew2bd, eb2t, nwbd, nbt, wabbd)


def _t3_body(s_ref, cnt_ref, invd_ref, ew2_ref, eb2_ref, nw_ref, nb_ref,
             ow1_ref, ob1_ref, ow2_ref, ob2_ref, out_ref):
    ssum = s_ref[0] + s_ref[1]
    agg = jnp.dot(ssum, ew2_ref[...], preferred_element_type=jnp.float32)
    agg = (agg + cnt_ref[...] * eb2_ref[...]) * invd_ref[...]
    nf = jnp.dot(agg, nw_ref[...], preferred_element_type=jnp.float32)
    nf = jnp.maximum(nf + nb_ref[...], 0.0)
    h = jnp.dot(nf, ow1_ref[...], preferred_element_type=jnp.float32)
    h = jnp.maximum(h + ob1_ref[...], 0.0)
    out_ref[...] = jnp.dot(h, ow2_ref[...],
                           preferred_element_type=jnp.float32) + ob2_ref[...]


def _node_final(s_parts, cnt, invd, ew2bd, eb2t, nwbd, nbt,
                ow1bd, ob1t, ow2bd, ob2t):
    return pl.pallas_call(
        _t3_body,
        grid=(NBLOCKS,),
        in_specs=[
            pl.BlockSpec((NC, NB, 128), lambda i: (0, i, 0)),
            pl.BlockSpec((NB, 128), lambda i: (i, 0)),
            pl.BlockSpec((NB, 128), lambda i: (i, 0)),
            pl.BlockSpec((128, 128), lambda i: (0, 0)),
            pl.BlockSpec((1, 128), lambda i: (0, 0)),
            pl.BlockSpec((128, 128), lambda i: (0, 0)),
            pl.BlockSpec((1, 128), lambda i: (0, 0)),
            pl.BlockSpec((128, 128), lambda i: (0, 0)),
            pl.BlockSpec((1, 128), lambda i: (0, 0)),
            pl.BlockSpec((128, 128), lambda i: (0, 0)),
            pl.BlockSpec((1, 128), lambda i: (0, 0)),
        ],
        out_specs=pl.BlockSpec((NB, 128), lambda i: (i, 0)),
        out_shape=_sds((NR, 128)),
    )(s_parts, cnt, invd, ew2bd, eb2t, nwbd, nbt, ow1bd, ob1t, ow2bd, ob2t)


def _bd(w):
    return jnp.kron(jnp.eye(8, dtype=jnp.float32), w)


def _tile8(b):
    return jnp.tile(b, 8).reshape(1, 128)


def kernel(x, edge_index, edge_attr,
           e2n_W1, e2n_b1, e2n_W2, e2n_b2,
           l0_eW1, l0_eb1, l0_eW2, l0_eb2, l0_nW, l0_nb,
           l1_eW1, l1_eb1, l1_eW2, l1_eb2, l1_nW, l1_nb,
           out_W1, out_b1, out_W2, out_b2):
    ei3 = edge_index.reshape(2, IDX_ROWS, SUB)
    ea_t = edge_attr.T

    wcat_t = jnp.concatenate(
        [e2n_W1, l0_eW1[2 * H:], l1_eW1[2 * H:]], axis=1).T
    bcol = jnp.concatenate([e2n_b1, l0_eb1, l1_eb1]).reshape(3 * H, 1)
    a0t, r0t, r1t = _edge_precompute(ea_t, wcat_t, bcol)

    s0_parts, cnt_parts = _sc_scatter0(ei3, a0t)

    wab0 = jnp.concatenate([_bd(l0_eW1[:H]), _bd(l0_eW1[H:2 * H])], axis=1)
    p0, q0, cnt, invd = _node_init(
        s0_parts.reshape(NC, NR, 128), cnt_parts.reshape(NC, NR, 128),
        _bd(e2n_W2), _tile8(e2n_b2), wab0)

    s_parts = _sc_layer(ei3, p0.reshape(N, H), q0.reshape(N, H), r0t)
    wab1 = jnp.concatenate([_bd(l1_eW1[:H]), _bd(l1_eW1[H:2 * H])], axis=1)
    p1, q1 = _node_update(s_parts.reshape(NC, NR, 128), cnt, invd,
                          _bd(l0_eW2), _tile8(l0_eb2),
                          _bd(l0_nW), _tile8(l0_nb), wab1)

    s_parts = _sc_layer(ei3, p1.reshape(N, H), q1.reshape(N, H), r1t)
    pred = _node_final(s_parts.reshape(NC, NR, 128), cnt, invd,
                       _bd(l1_eW2), _tile8(l1_eb2),
                       _bd(l1_nW), _tile8(l1_nb),
                       _bd(out_W1), _tile8(out_b1),
                       _bd(out_W2), _tile8(out_b2))
    return pred.reshape(N, H)


# speedup vs baseline: 3.1837x; 3.1837x over previous
import functools

import jax
import jax.numpy as jnp
from jax import lax
from jax.experimental import pallas as pl
from jax.experimental.pallas import tpu as pltpu
from jax.experimental.pallas import tpu_sc as plsc

N = 100000
E = 1600000
H = 16

NC = 2
NS = 16
NW = NC * NS

SUB = 128
GRP = 2
CHUNK = SUB * GRP
NCH = E // CHUNK
CH_HI = 196
CH_LO = 194
W_HI = 21
IDX_ROWS = E // SUB

RPT = N // NS
ZROWS = 125
ZCOPIES = RPT // ZROWS

_mesh = plsc.VectorSubcoreMesh(core_axis_name="c", subcore_axis_name="s",
                               num_cores=NC, num_subcores=NS)
_sc_params = pltpu.CompilerParams(use_tc_tiling_on_sc=False)


def _sds(shape):
    return jax.ShapeDtypeStruct(shape, jnp.float32)


def _worker_chunks(wid):
    nb = jnp.where(wid < W_HI, CH_HI, CH_LO)
    cbase = jnp.where(wid < W_HI, wid * CH_HI,
                      W_HI * CH_HI + (wid - W_HI) * CH_LO)
    return nb, cbase


def _zero_acc(acc, zb, sem, my_rows):
    ds = [pltpu.async_copy(zb, acc.at[pl.ds(my_rows + j * ZROWS, ZROWS)], sem)
          for j in range(ZCOPIES)]
    for d in ds:
        d.wait()


@functools.partial(
    pl.kernel,
    out_type=(_sds((NC, N, H)), _sds((NC, N, H))),
    mesh=_mesh,
    scratch_types=[
        pltpu.VMEM((GRP, SUB), jnp.int32),
        pltpu.VMEM((CHUNK, H), jnp.float32),
        pltpu.VMEM((ZROWS, H), jnp.float32),
        pltpu.VMEM_SHARED((N, H), jnp.float32),
        pltpu.SemaphoreType.DMA,
    ],
    compiler_params=_sc_params,
)
def _sc_scatter0(ei_hbm, a0_hbm, s0_out, cnt_out, idxd, rows, zb, acc, sem):
    c = lax.axis_index("c")
    s = lax.axis_index("s")
    wid = s * NC + c
    nb, cbase = _worker_chunks(wid)
    my_rows = s * RPT
    my_slice = pl.ds(my_rows, RPT)

    @plsc.parallel_loop(0, ZROWS, unroll=8)
    def _(i):
        zb[i, :] = jnp.zeros((H,), jnp.float32)

    _zero_acc(acc, zb, sem, my_rows)
    plsc.subcore_barrier()

    def chunk_a(i, _):
        g = cbase + i
        pltpu.sync_copy(ei_hbm.at[1, pl.ds(g * GRP, GRP)], idxd)
        pltpu.async_copy(a0_hbm.at[pl.ds(g * CHUNK, CHUNK)], rows, sem).wait()
        for j in range(GRP):
            pltpu.sync_copy(rows.at[pl.ds(j * SUB, SUB)],
                            acc.at[idxd.at[j]], add=True)
        return 0

    lax.fori_loop(0, nb, chunk_a, 0)
    plsc.subcore_barrier()
    pltpu.sync_copy(acc.at[my_slice], s0_out.at[c, my_slice])
    plsc.subcore_barrier()

    _zero_acc(acc, zb, sem, my_rows)

    @plsc.parallel_loop(0, CHUNK, unroll=8)
    def _(i):
        rows[i, :] = jnp.ones((H,), jnp.float32)

    plsc.subcore_barrier()

    def chunk_b(i, _):
        g = cbase + i
        pltpu.sync_copy(ei_hbm.at[1, pl.ds(g * GRP, GRP)], idxd)
        for j in range(GRP):
            pltpu.sync_copy(rows.at[pl.ds(j * SUB, SUB)],
                            acc.at[idxd.at[j]], add=True)
        return 0

    lax.fori_loop(0, nb, chunk_b, 0)
    plsc.subcore_barrier()
    pltpu.sync_copy(acc.at[my_slice], cnt_out.at[c, my_slice])


_LAYER_SCRATCH = []
for _set in range(2):
    _LAYER_SCRATCH += [
        pltpu.VMEM((GRP, SUB), jnp.int32),
        pltpu.VMEM((GRP, SUB), jnp.int32),
        pltpu.VMEM((CHUNK, H), jnp.float32),
        pltpu.VMEM((CHUNK, H), jnp.float32),
        pltpu.VMEM((CHUNK, H), jnp.float32),
        pltpu.SemaphoreType.DMA,
        pltpu.SemaphoreType.DMA,
    ]
_LAYER_SCRATCH += [
    pltpu.VMEM((ZROWS, H), jnp.float32),
    pltpu.VMEM_SHARED((N, H), jnp.float32),
    pltpu.SemaphoreType.DMA,
]


@functools.partial(
    pl.kernel,
    out_type=_sds((NC, N, H)),
    mesh=_mesh,
    scratch_types=_LAYER_SCRATCH,
    compiler_params=_sc_params,
)
def _sc_layer(ei_hbm, p_hbm, q_hbm, r_hbm, s_out,
              idxd0, idxs0, pb0, qb0, rb0, gsem0, ssem0,
              idxd1, idxs1, pb1, qb1, rb1, gsem1, ssem1,
              zb, acc, zsem):
    c = lax.axis_index("c")
    s = lax.axis_index("s")
    wid = s * NC + c
    nb, cbase = _worker_chunks(wid)
    my_rows = s * RPT
    my_slice = pl.ds(my_rows, RPT)
    sets = ((idxd0, idxs0, pb0, qb0, rb0, gsem0, ssem0),
            (idxd1, idxs1, pb1, qb1, rb1, gsem1, ssem1))

    @plsc.parallel_loop(0, ZROWS, unroll=8)
    def _(i):
        zb[i, :] = jnp.zeros((H,), jnp.float32)

    _zero_acc(acc, zb, zsem, my_rows)
    plsc.subcore_barrier()

    def fetch(g, st):
        idxd, idxs, pb, qb, rb, gsem, _ = st
        pltpu.sync_copy(ei_hbm.at[1, pl.ds(g * GRP, GRP)], idxd)
        pltpu.sync_copy(ei_hbm.at[0, pl.ds(g * GRP, GRP)], idxs)
        pltpu.async_copy(r_hbm.at[pl.ds(g * CHUNK, CHUNK)], rb, gsem)
        for j in range(GRP):
            sl = pl.ds(j * SUB, SUB)
            pltpu.async_copy(p_hbm.at[idxd.at[j]], pb.at[sl], gsem)
            pltpu.async_copy(q_hbm.at[idxs.at[j]], qb.at[sl], gsem)

    def drain(sem, donor, times):
        for _ in range(times):
            pltpu.make_async_copy(r_hbm.at[pl.ds(0, CHUNK)], donor, sem).wait()

    def process(st):
        idxd, _, pb, qb, rb, gsem, ssem = st
        drain(gsem, pb, 3)

        @plsc.parallel_loop(0, CHUNK, unroll=8)
        def _(k):
            pb[k, :] = jnp.maximum(pb[k, :] + qb[k, :] + rb[k, :], 0.0)

        for j in range(GRP):
            pltpu.async_copy(pb.at[pl.ds(j * SUB, SUB)],
                             acc.at[idxd.at[j]], ssem, add=True)

    fetch(cbase, sets[0])
    fetch(cbase + 1, sets[1])

    def pair(t, _):
        for sl in range(2):
            st = sets[sl]
            g = 2 * t + sl
            process(st)
            nxt = g + 2

            @pl.when(nxt < nb)
            def _():
                drain(st[6], st[2], 1)
                fetch(cbase + nxt, st)
        return 0

    lax.fori_loop(0, nb // 2, pair, 0)
    drain(ssem0, pb0, 1)
    drain(ssem1, pb1, 1)
    plsc.subcore_barrier()
    pltpu.sync_copy(acc.at[my_slice], s_out.at[c, my_slice])


ER = E // 8
EB = 1000
EBLOCKS = ER // EB
NR = N // 8
NB = 512
NBLOCKS = -(-NR // NB)


def _t0_body(ea_ref, w_ref, b_ref, a0_ref, r0_ref, r1_ref):
    y = jnp.dot(ea_ref[...], w_ref[...], preferred_element_type=jnp.float32)
    y = y + b_ref[...]
    a0_ref[...] = jnp.maximum(y[:, :128], 0.0)
    r0_ref[...] = y[:, 128:256]
    r1_ref[...] = y[:, 256:]


def _edge_precompute(ea_r, wbd, btile):
    return pl.pallas_call(
        _t0_body,
        grid=(EBLOCKS,),
        in_specs=[
            pl.BlockSpec((EB, 128), lambda i: (i, 0)),
            pl.BlockSpec((128, 384), lambda i: (0, 0)),
            pl.BlockSpec((1, 384), lambda i: (0, 0)),
        ],
        out_specs=[pl.BlockSpec((EB, 128), lambda i: (i, 0))] * 3,
        out_shape=[_sds((ER, 128))] * 3,
    )(ea_r, wbd, btile)


def _t1_body(s_ref, c_ref, w2_ref, b2_ref, wab_ref, p_ref, q_ref,
             cnt_ref, invd_ref):
    ssum = s_ref[0] + s_ref[1]
    cnt = c_ref[0] + c_ref[1]
    nf = jnp.dot(ssum, w2_ref[...], preferred_element_type=jnp.float32)
    nf = nf + cnt * b2_ref[...]
    pq = jnp.dot(nf, wab_ref[...], preferred_element_type=jnp.float32)
    p_ref[...] = pq[:, :128]
    q_ref[...] = pq[:, 128:]
    cnt_ref[...] = cnt
    invd_ref[...] = 1.0 / jnp.maximum(cnt, 1.0)


def _node_init(s0_parts, cnt_parts, w2bd, b2tile, wabbd):
    return pl.pallas_call(
        _t1_body,
        grid=(NBLOCKS,),
        in_specs=[
            pl.BlockSpec((NC, NB, 128), lambda i: (0, i, 0)),
            pl.BlockSpec((NC, NB, 128), lambda i: (0, i, 0)),
            pl.BlockSpec((128, 128), lambda i: (0, 0)),
            pl.BlockSpec((1, 128), lambda i: (0, 0)),
            pl.BlockSpec((128, 256), lambda i: (0, 0)),
        ],
        out_specs=[pl.BlockSpec((NB, 128), lambda i: (i, 0))] * 4,
        out_shape=[_sds((NR, 128))] * 4,
    )(s0_parts, cnt_parts, w2bd, b2tile, wabbd)


def _t2_body(s_ref, cnt_ref, invd_ref, ew2_ref, eb2_ref, nw_ref, nb_ref,
             wab_ref, p_ref, q_ref):
    ssum = s_ref[0] + s_ref[1]
    agg = jnp.dot(ssum, ew2_ref[...], preferred_element_type=jnp.float32)
    agg = (agg + cnt_ref[...] * eb2_ref[...]) * invd_ref[...]
    nf = jnp.dot(agg, nw_ref[...], preferred_element_type=jnp.float32)
    nf = jnp.maximum(nf + nb_ref[...], 0.0)
    pq = jnp.dot(nf, wab_ref[...], preferred_element_type=jnp.float32)
    p_ref[...] = pq[:, :128]
    q_ref[...] = pq[:, 128:]


def _node_update(s_parts, cnt, invd, ew2bd, eb2t, nwbd, nbt, wabbd):
    return pl.pallas_call(
        _t2_body,
        grid=(NBLOCKS,),
        in_specs=[
            pl.BlockSpec((NC, NB, 128), lambda i: (0, i, 0)),
            pl.BlockSpec((NB, 128), lambda i: (i, 0)),
            pl.BlockSpec((NB, 128), lambda i: (i, 0)),
            pl.BlockSpec((128, 128), lambda i: (0, 0)),
            pl.BlockSpec((1, 128), lambda i: (0, 0)),
            pl.BlockSpec((128, 128), lambda i: (0, 0)),
            pl.BlockSpec((1, 128), lambda i: (0, 0)),
            pl.BlockSpec((128, 256), lambda i: (0, 0)),
        ],
        out_specs=[pl.BlockSpec((NB, 128), lambda i: (i, 0))] * 2,
        out_shape=[_sds((NR, 128))] * 2,
    )(s_parts, cnt, invd, ew2bd, eb2t, nwbd, nbt, wabbd)


def _t3_body(s_ref, cnt_ref, invd_ref, ew2_ref, eb2_ref, nw_ref, nb_ref,
             ow1_ref, ob1_ref, ow2_ref, ob2_ref, out_ref):
    ssum = s_ref[0] + s_ref[1]
    agg = jnp.dot(ssum, ew2_ref[...], preferred_element_type=jnp.float32)
    agg = (agg + cnt_ref[...] * eb2_ref[...]) * invd_ref[...]
    nf = jnp.dot(agg, nw_ref[...], preferred_element_type=jnp.float32)
    nf = jnp.maximum(nf + nb_ref[...], 0.0)
    h = jnp.dot(nf, ow1_ref[...], preferred_element_type=jnp.float32)
    h = jnp.maximum(h + ob1_ref[...], 0.0)
    out_ref[...] = jnp.dot(h, ow2_ref[...],
                           preferred_element_type=jnp.float32) + ob2_ref[...]


def _node_final(s_parts, cnt, invd, ew2bd, eb2t, nwbd, nbt,
                ow1bd, ob1t, ow2bd, ob2t):
    return pl.pallas_call(
        _t3_body,
        grid=(NBLOCKS,),
        in_specs=[
            pl.BlockSpec((NC, NB, 128), lambda i: (0, i, 0)),
            pl.BlockSpec((NB, 128), lambda i: (i, 0)),
            pl.BlockSpec((NB, 128), lambda i: (i, 0)),
            pl.BlockSpec((128, 128), lambda i: (0, 0)),
            pl.BlockSpec((1, 128), lambda i: (0, 0)),
            pl.BlockSpec((128, 128), lambda i: (0, 0)),
            pl.BlockSpec((1, 128), lambda i: (0, 0)),
            pl.BlockSpec((128, 128), lambda i: (0, 0)),
            pl.BlockSpec((1, 128), lambda i: (0, 0)),
            pl.BlockSpec((128, 128), lambda i: (0, 0)),
            pl.BlockSpec((1, 128), lambda i: (0, 0)),
        ],
        out_specs=pl.BlockSpec((NB, 128), lambda i: (i, 0)),
        out_shape=_sds((NR, 128)),
    )(s_parts, cnt, invd, ew2bd, eb2t, nwbd, nbt, ow1bd, ob1t, ow2bd, ob2t)


def _bd(w):
    return jnp.kron(jnp.eye(8, dtype=jnp.float32), w)


def _tile8(b):
    return jnp.tile(b, 8).reshape(1, 128)


def kernel(x, edge_index, edge_attr,
           e2n_W1, e2n_b1, e2n_W2, e2n_b2,
           l0_eW1, l0_eb1, l0_eW2, l0_eb2, l0_nW, l0_nb,
           l1_eW1, l1_eb1, l1_eW2, l1_eb2, l1_nW, l1_nb,
           out_W1, out_b1, out_W2, out_b2):
    ei3 = edge_index.reshape(2, IDX_ROWS, SUB)
    ea_r = edge_attr.reshape(ER, 128)

    wbd = jnp.concatenate(
        [_bd(e2n_W1), _bd(l0_eW1[2 * H:]), _bd(l1_eW1[2 * H:])], axis=1)
    btile = jnp.concatenate(
        [_tile8(e2n_b1), _tile8(l0_eb1), _tile8(l1_eb1)], axis=1)
    a0, r0, r1 = _edge_precompute(ea_r, wbd, btile)

    s0_parts, cnt_parts = _sc_scatter0(ei3, a0.reshape(E, H))

    wab0 = jnp.concatenate([_bd(l0_eW1[:H]), _bd(l0_eW1[H:2 * H])], axis=1)
    p0, q0, cnt, invd = _node_init(
        s0_parts.reshape(NC, NR, 128), cnt_parts.reshape(NC, NR, 128),
        _bd(e2n_W2), _tile8(e2n_b2), wab0)

    s_parts = _sc_layer(ei3, p0.reshape(N, H), q0.reshape(N, H),
                        r0.reshape(E, H))
    wab1 = jnp.concatenate([_bd(l1_eW1[:H]), _bd(l1_eW1[H:2 * H])], axis=1)
    p1, q1 = _node_update(s_parts.reshape(NC, NR, 128), cnt, invd,
                          _bd(l0_eW2), _tile8(l0_eb2),
                          _bd(l0_nW), _tile8(l0_nb), wab1)

    s_parts = _sc_layer(ei3, p1.reshape(N, H), q1.reshape(N, H),
                        r1.reshape(E, H))
    pred = _node_final(s_parts.reshape(NC, NR, 128), cnt, invd,
                       _bd(l1_eW2), _tile8(l1_eb2),
                       _bd(l1_nW), _tile8(l1_nb),
                       _bd(out_W1), _tile8(out_b1),
                       _bd(out_W2), _tile8(out_b2))
    return pred.reshape(N, H)


# speedup vs baseline: 4.0318x; 1.2664x over previous
import functools

import jax
import jax.numpy as jnp
from jax import lax
from jax.experimental import pallas as pl
from jax.experimental.pallas import tpu as pltpu
from jax.experimental.pallas import tpu_sc as plsc

N = 100000
E = 1600000
H = 16

NC = 2
NS = 16
NW = NC * NS

SUB = 128
GRP = 2
CHUNK = SUB * GRP
NCH = E // CHUNK
CH_HI = 196
CH_LO = 194
W_HI = 21
IDX_ROWS = E // SUB

RPT = N // NS
ZROWS = 125
ZCOPIES = RPT // ZROWS

_mesh = plsc.VectorSubcoreMesh(core_axis_name="c", subcore_axis_name="s",
                               num_cores=NC, num_subcores=NS)
_sc_params = pltpu.CompilerParams(use_tc_tiling_on_sc=False)


def _sds(shape):
    return jax.ShapeDtypeStruct(shape, jnp.float32)


def _worker_chunks(wid):
    nb = jnp.where(wid < W_HI, CH_HI, CH_LO)
    cbase = jnp.where(wid < W_HI, wid * CH_HI,
                      W_HI * CH_HI + (wid - W_HI) * CH_LO)
    return nb, cbase


def _zero_acc(acc, zb, sem, my_rows):
    ds = [pltpu.async_copy(zb, acc.at[pl.ds(my_rows + j * ZROWS, ZROWS)], sem)
          for j in range(ZCOPIES)]
    for d in ds:
        d.wait()


S_GRP = 4
S_CHUNK = SUB * S_GRP
S_NCH = E // S_CHUNK
S_HI = 98
S_LO = 97


def _s0_chunks(wid):
    nb = jnp.where(wid < W_HI, S_HI, S_LO)
    cbase = jnp.where(wid < W_HI, wid * S_HI,
                      W_HI * S_HI + (wid - W_HI) * S_LO)
    return nb, cbase


_S0_SCRATCH = []
for _set in range(2):
    _S0_SCRATCH += [
        pltpu.VMEM((S_GRP, SUB), jnp.int32),
        pltpu.VMEM((S_CHUNK, H), jnp.float32),
        pltpu.SemaphoreType.DMA,
        pltpu.SemaphoreType.DMA,
    ]
_S0_SCRATCH += [
    pltpu.VMEM((ZROWS, H), jnp.float32),
    pltpu.VMEM_SHARED((N, H), jnp.float32),
    pltpu.SemaphoreType.DMA,
]


@functools.partial(
    pl.kernel,
    out_type=(_sds((NC, N, H)), _sds((NC, N, H))),
    mesh=_mesh,
    scratch_types=_S0_SCRATCH,
    compiler_params=_sc_params,
)
def _sc_scatter0(ei_hbm, a0_hbm, s0_out, cnt_out,
                 idx0, rows0, gsem0, ssem0,
                 idx1, rows1, gsem1, ssem1,
                 zb, acc, zsem):
    c = lax.axis_index("c")
    s = lax.axis_index("s")
    wid = s * NC + c
    nb, cbase = _s0_chunks(wid)
    my_rows = s * RPT
    my_slice = pl.ds(my_rows, RPT)
    sets = ((idx0, rows0, gsem0, ssem0), (idx1, rows1, gsem1, ssem1))

    @plsc.parallel_loop(0, ZROWS, unroll=8)
    def _(i):
        zb[i, :] = jnp.zeros((H,), jnp.float32)

    _zero_acc(acc, zb, zsem, my_rows)
    plsc.subcore_barrier()

    def fetch(g, st):
        idx, rows, gsem, _ = st
        pltpu.sync_copy(ei_hbm.at[1, pl.ds(g * S_GRP, S_GRP)], idx)
        pltpu.async_copy(a0_hbm.at[pl.ds(g * S_CHUNK, S_CHUNK)], rows, gsem)

    def drain(sem, donor):
        pltpu.make_async_copy(a0_hbm.at[pl.ds(0, S_CHUNK)], donor, sem).wait()

    fetch(cbase, sets[0])

    @pl.when(nb > 1)
    def _():
        fetch(cbase + 1, sets[1])

    def pair_a(t, _):
        for sl in range(2):
            st = sets[sl]
            g = 2 * t + sl

            @pl.when(g < nb)
            def _():
                idx, rows, gsem, ssem = st
                drain(gsem, rows)
                for j in range(S_GRP):
                    pltpu.async_copy(rows.at[pl.ds(j * SUB, SUB)],
                                     acc.at[idx.at[j]], ssem, add=True)
                drain(ssem, rows)

                @pl.when(g + 2 < nb)
                def _():
                    fetch(cbase + g + 2, st)
        return 0

    lax.fori_loop(0, (S_HI + 1) // 2, pair_a, 0)
    plsc.subcore_barrier()
    pltpu.sync_copy(acc.at[my_slice], s0_out.at[c, my_slice])
    plsc.subcore_barrier()

    _zero_acc(acc, zb, zsem, my_rows)

    @plsc.parallel_loop(0, S_CHUNK, unroll=8)
    def _(i):
        rows0[i, :] = jnp.ones((H,), jnp.float32)

    plsc.subcore_barrier()

    def pair_b(t, _):
        for sl in range(2):
            st = sets[sl]
            g = 2 * t + sl

            @pl.when(g < nb)
            def _():
                idx, _, _, ssem = st

                @pl.when(g >= 2)
                def _():
                    drain(ssem, rows1)
                pltpu.sync_copy(ei_hbm.at[1, pl.ds((cbase + g) * S_GRP,
                                                   S_GRP)], idx)
                for j in range(S_GRP):
                    pltpu.async_copy(rows0.at[pl.ds(j * SUB, SUB)],
                                     acc.at[idx.at[j]], ssem, add=True)
        return 0

    lax.fori_loop(0, (S_HI + 1) // 2, pair_b, 0)
    drain(ssem0, rows1)
    drain(ssem1, rows1)
    plsc.subcore_barrier()
    pltpu.sync_copy(acc.at[my_slice], cnt_out.at[c, my_slice])


_LAYER_SCRATCH = []
for _set in range(2):
    _LAYER_SCRATCH += [
        pltpu.VMEM((2, GRP, SUB), jnp.int32),
        pltpu.VMEM((CHUNK, H), jnp.float32),
        pltpu.VMEM((CHUNK, H), jnp.float32),
        pltpu.VMEM((CHUNK, H), jnp.float32),
        pltpu.SemaphoreType.DMA,
        pltpu.SemaphoreType.DMA,
    ]
_LAYER_SCRATCH += [
    pltpu.VMEM((ZROWS, H), jnp.float32),
    pltpu.VMEM_SHARED((N, H), jnp.float32),
    pltpu.SemaphoreType.DMA,
]


@functools.partial(
    pl.kernel,
    out_type=_sds((NC, N, H)),
    mesh=_mesh,
    scratch_types=_LAYER_SCRATCH,
    compiler_params=_sc_params,
)
def _sc_layer(ei_hbm, p_hbm, q_hbm, r_hbm, s_out,
              idx0, pb0, qb0, rb0, gsem0, ssem0,
              idx1, pb1, qb1, rb1, gsem1, ssem1,
              zb, acc, zsem):
    c = lax.axis_index("c")
    s = lax.axis_index("s")
    wid = s * NC + c
    nb, cbase = _worker_chunks(wid)
    my_rows = s * RPT
    my_slice = pl.ds(my_rows, RPT)
    sets = ((idx0, pb0, qb0, rb0, gsem0, ssem0),
            (idx1, pb1, qb1, rb1, gsem1, ssem1))

    @plsc.parallel_loop(0, ZROWS, unroll=8)
    def _(i):
        zb[i, :] = jnp.zeros((H,), jnp.float32)

    _zero_acc(acc, zb, zsem, my_rows)
    plsc.subcore_barrier()

    def fetch(g, st):
        idx, pb, qb, rb, gsem, _ = st
        pltpu.sync_copy(ei_hbm.at[:, pl.ds(g * GRP, GRP)], idx)
        pltpu.async_copy(r_hbm.at[pl.ds(g * CHUNK, CHUNK)], rb, gsem)
        for j in range(GRP):
            sl = pl.ds(j * SUB, SUB)
            pltpu.async_copy(p_hbm.at[idx.at[1, j]], pb.at[sl], gsem)
            pltpu.async_copy(q_hbm.at[idx.at[0, j]], qb.at[sl], gsem)

    def drain(sem, donor, times):
        for _ in range(times):
            pltpu.make_async_copy(r_hbm.at[pl.ds(0, CHUNK)], donor, sem).wait()

    def process(st):
        idx, pb, qb, rb, gsem, ssem = st
        drain(gsem, pb, 3)

        @plsc.parallel_loop(0, CHUNK, unroll=8)
        def _(k):
            pb[k, :] = jnp.maximum(pb[k, :] + qb[k, :] + rb[k, :], 0.0)

        for j in range(GRP):
            pltpu.async_copy(pb.at[pl.ds(j * SUB, SUB)],
                             acc.at[idx.at[1, j]], ssem, add=True)

    fetch(cbase, sets[0])
    fetch(cbase + 1, sets[1])

    def pair(t, _):
        for sl in range(2):
            st = sets[sl]
            g = 2 * t + sl
            process(st)
            nxt = g + 2

            @pl.when(nxt < nb)
            def _():
                drain(st[5], st[1], 1)
                fetch(cbase + nxt, st)
        return 0

    lax.fori_loop(0, nb // 2, pair, 0)
    drain(ssem0, pb0, 1)
    drain(ssem1, pb1, 1)
    plsc.subcore_barrier()
    pltpu.sync_copy(acc.at[my_slice], s_out.at[c, my_slice])


ER = E // 8
EB = 1000
EBLOCKS = ER // EB
NR = N // 8
NB = 512
NBLOCKS = -(-NR // NB)


def _t0_body(ea_ref, w_ref, b_ref, a0_ref, r0_ref, r1_ref):
    y = jnp.dot(ea_ref[...], w_ref[...], preferred_element_type=jnp.float32)
    y = y + b_ref[...]
    a0_ref[...] = jnp.maximum(y[:, :128], 0.0)
    r0_ref[...] = y[:, 128:256]
    r1_ref[...] = y[:, 256:]


def _edge_precompute(ea_r, wbd, btile):
    return pl.pallas_call(
        _t0_body,
        grid=(EBLOCKS,),
        in_specs=[
            pl.BlockSpec((EB, 128), lambda i: (i, 0)),
            pl.BlockSpec((128, 384), lambda i: (0, 0)),
            pl.BlockSpec((1, 384), lambda i: (0, 0)),
        ],
        out_specs=[pl.BlockSpec((EB, 128), lambda i: (i, 0))] * 3,
        out_shape=[_sds((ER, 128))] * 3,
    )(ea_r, wbd, btile)


def _t1_body(s_ref, c_ref, w2_ref, b2_ref, wab_ref, p_ref, q_ref,
             cnt_ref, invd_ref):
    ssum = s_ref[0] + s_ref[1]
    cnt = c_ref[0] + c_ref[1]
    nf = jnp.dot(ssum, w2_ref[...], preferred_element_type=jnp.float32)
    nf = nf + cnt * b2_ref[...]
    pq = jnp.dot(nf, wab_ref[...], preferred_element_type=jnp.float32)
    p_ref[...] = pq[:, :128]
    q_ref[...] = pq[:, 128:]
    cnt_ref[...] = cnt
    invd_ref[...] = 1.0 / jnp.maximum(cnt, 1.0)


def _node_init(s0_parts, cnt_parts, w2bd, b2tile, wabbd):
    return pl.pallas_call(
        _t1_body,
        grid=(NBLOCKS,),
        in_specs=[
            pl.BlockSpec((NC, NB, 128), lambda i: (0, i, 0)),
            pl.BlockSpec((NC, NB, 128), lambda i: (0, i, 0)),
            pl.BlockSpec((128, 128), lambda i: (0, 0)),
            pl.BlockSpec((1, 128), lambda i: (0, 0)),
            pl.BlockSpec((128, 256), lambda i: (0, 0)),
        ],
        out_specs=[pl.BlockSpec((NB, 128), lambda i: (i, 0))] * 4,
        out_shape=[_sds((NR, 128))] * 4,
    )(s0_parts, cnt_parts, w2bd, b2tile, wabbd)


def _t2_body(s_ref, cnt_ref, invd_ref, ew2_ref, eb2_ref, nw_ref, nb_ref,
             wab_ref, p_ref, q_ref):
    ssum = s_ref[0] + s_ref[1]
    agg = jnp.dot(ssum, ew2_ref[...], preferred_element_type=jnp.float32)
    agg = (agg + cnt_ref[...] * eb2_ref[...]) * invd_ref[...]
    nf = jnp.dot(agg, nw_ref[...], preferred_element_type=jnp.float32)
    nf = jnp.maximum(nf + nb_ref[...], 0.0)
    pq = jnp.dot(nf, wab_ref[...], preferred_element_type=jnp.float32)
    p_ref[...] = pq[:, :128]
    q_ref[...] = pq[:, 128:]


def _node_update(s_parts, cnt, invd, ew2bd, eb2t, nwbd, nbt, wabbd):
    return pl.pallas_call(
        _t2_body,
        grid=(NBLOCKS,),
        in_specs=[
            pl.BlockSpec((NC, NB, 128), lambda i: (0, i, 0)),
            pl.BlockSpec((NB, 128), lambda i: (i, 0)),
            pl.BlockSpec((NB, 128), lambda i: (i, 0)),
            pl.BlockSpec((128, 128), lambda i: (0, 0)),
            pl.BlockSpec((1, 128), lambda i: (0, 0)),
            pl.BlockSpec((128, 128), lambda i: (0, 0)),
            pl.BlockSpec((1, 128), lambda i: (0, 0)),
            pl.BlockSpec((128, 256), lambda i: (0, 0)),
        ],
        out_specs=[pl.BlockSpec((NB, 128), lambda i: (i, 0))] * 2,
        out_shape=[_sds((NR, 128))] * 2,
    )(s_parts, cnt, invd, ew2bd, eb2t, nwbd, nbt, wabbd)


def _t3_body(s_ref, cnt_ref, invd_ref, ew2_ref, eb2_ref, nw_ref, nb_ref,
             ow1_ref, ob1_ref, ow2_ref, ob2_ref, out_ref):
    ssum = s_ref[0] + s_ref[1]
    agg = jnp.dot(ssum, ew2_ref[...], preferred_element_type=jnp.float32)
    agg = (agg + cnt_ref[...] * eb2_ref[...]) * invd_ref[...]
    nf = jnp.dot(agg, nw_ref[...], preferred_element_type=jnp.float32)
    nf = jnp.maximum(nf + nb_ref[...], 0.0)
    h = jnp.dot(nf, ow1_ref[...], preferred_element_type=jnp.float32)
    h = jnp.maximum(h + ob1_ref[...], 0.0)
    out_ref[...] = jnp.dot(h, ow2_ref[...],
                           preferred_element_type=jnp.float32) + ob2_ref[...]


def _node_final(s_parts, cnt, invd, ew2bd, eb2t, nwbd, nbt,
                ow1bd, ob1t, ow2bd, ob2t):
    return pl.pallas_call(
        _t3_body,
        grid=(NBLOCKS,),
        in_specs=[
            pl.BlockSpec((NC, NB, 128), lambda i: (0, i, 0)),
            pl.BlockSpec((NB, 128), lambda i: (i, 0)),
            pl.BlockSpec((NB, 128), lambda i: (i, 0)),
            pl.BlockSpec((128, 128), lambda i: (0, 0)),
            pl.BlockSpec((1, 128), lambda i: (0, 0)),
            pl.BlockSpec((128, 128), lambda i: (0, 0)),
            pl.BlockSpec((1, 128), lambda i: (0, 0)),
            pl.BlockSpec((128, 128), lambda i: (0, 0)),
            pl.BlockSpec((1, 128), lambda i: (0, 0)),
            pl.BlockSpec((128, 128), lambda i: (0, 0)),
            pl.BlockSpec((1, 128), lambda i: (0, 0)),
        ],
        out_specs=pl.BlockSpec((NB, 128), lambda i: (i, 0)),
        out_shape=_sds((NR, 128)),
    )(s_parts, cnt, invd, ew2bd, eb2t, nwbd, nbt, ow1bd, ob1t, ow2bd, ob2t)


def _bd(w):
    return jnp.kron(jnp.eye(8, dtype=jnp.float32), w)


def _tile8(b):
    return jnp.tile(b, 8).reshape(1, 128)


def kernel(x, edge_index, edge_attr,
           e2n_W1, e2n_b1, e2n_W2, e2n_b2,
           l0_eW1, l0_eb1, l0_eW2, l0_eb2, l0_nW, l0_nb,
           l1_eW1, l1_eb1, l1_eW2, l1_eb2, l1_nW, l1_nb,
           out_W1, out_b1, out_W2, out_b2):
    ei3 = edge_index.reshape(2, IDX_ROWS, SUB)
    ea_r = edge_attr.reshape(ER, 128)

    wbd = jnp.concatenate(
        [_bd(e2n_W1), _bd(l0_eW1[2 * H:]), _bd(l1_eW1[2 * H:])], axis=1)
    btile = jnp.concatenate(
        [_tile8(e2n_b1), _tile8(l0_eb1), _tile8(l1_eb1)], axis=1)
    a0, r0, r1 = _edge_precompute(ea_r, wbd, btile)

    s0_parts, cnt_parts = _sc_scatter0(ei3, a0.reshape(E, H))

    wab0 = jnp.concatenate([_bd(l0_eW1[:H]), _bd(l0_eW1[H:2 * H])], axis=1)
    p0, q0, cnt, invd = _node_init(
        s0_parts.reshape(NC, NR, 128), cnt_parts.reshape(NC, NR, 128),
        _bd(e2n_W2), _tile8(e2n_b2), wab0)

    s_parts = _sc_layer(ei3, p0.reshape(N, H), q0.reshape(N, H),
                        r0.reshape(E, H))
    wab1 = jnp.concatenate([_bd(l1_eW1[:H]), _bd(l1_eW1[H:2 * H])], axis=1)
    p1, q1 = _node_update(s_parts.reshape(NC, NR, 128), cnt, invd,
                          _bd(l0_eW2), _tile8(l0_eb2),
                          _bd(l0_nW), _tile8(l0_nb), wab1)

    s_parts = _sc_layer(ei3, p1.reshape(N, H), q1.reshape(N, H),
                        r1.reshape(E, H))
    pred = _node_final(s_parts.reshape(NC, NR, 128), cnt, invd,
                       _bd(l1_eW2), _tile8(l1_eb2),
                       _bd(l1_nW), _tile8(l1_nb),
                       _bd(out_W1), _tile8(out_b1),
                       _bd(out_W2), _tile8(out_b2))
    return pred.reshape(N, H)
